# baseline (speedup 1.0000x reference)
import jax
import jax.numpy as jnp
from jax.experimental import pallas as pl
from jax.experimental.pallas import tpu as pltpu

_NUM_P = 8
_OUT_C = 64
_B = 16
_LOOK = 4
_NU = 6
_NO = 3


def _routing(u, wp_ref, bp_ref, wr_ref, br_ref):
    ones_row = jnp.ones((1, u.shape[0]), jnp.float32)
    pooled2 = jnp.dot(ones_row, u, preferred_element_type=jnp.float32)
    pooled = (pooled2[:, :_OUT_C] + pooled2[:, _OUT_C:]) * (
        1.0 / (2 * u.shape[0])
    )
    logits = (
        jnp.dot(pooled, wr_ref[...], preferred_element_type=jnp.float32)
        + br_ref[...]
    )
    iota = jax.lax.broadcasted_iota(jnp.int32, (1, _NUM_P), 1)
    m1 = jnp.max(logits, axis=1, keepdims=True)
    i1 = jnp.min(jnp.where(logits == m1, iota, _NUM_P), axis=1, keepdims=True)
    masked = jnp.where(iota == i1, -jnp.inf, logits)
    m2 = jnp.max(masked, axis=1, keepdims=True)
    i2 = jnp.min(jnp.where(masked == m2, iota, _NUM_P), axis=1, keepdims=True)
    e = jnp.exp(m2 - m1)
    p1 = 1.0 / (1.0 + e)
    p2 = e / (1.0 + e)
    acc = jnp.eye(_OUT_C, dtype=jnp.float32)
    bacc = jnp.zeros((1, _OUT_C), jnp.float32)
    for p in range(_NUM_P):
        w_p = jnp.where(i1 == p, p1, 0.0) + jnp.where(i2 == p, p2, 0.0)
        acc = acc + w_p * wp_ref[p]
        bacc = bacc + w_p * bp_ref[p : p + 1, :]
    z = jnp.zeros((_OUT_C, _OUT_C), jnp.float32)
    w2 = jnp.concatenate(
        [
            jnp.concatenate([acc, z], axis=1),
            jnp.concatenate([z, acc], axis=1),
        ],
        axis=0,
    )
    b2 = jnp.concatenate([bacc, bacc], axis=1)
    return w2, b2


def _body(u_hbm, wp_ref, bp_ref, wr_ref, br_ref, out_hbm,
          ubuf, obuf, in_sems, out_sems):
    u_pair = u_hbm
    o_pair = out_hbm

    def load(b):
        return pltpu.make_async_copy(
            u_pair.at[b], ubuf.at[b % _NU], in_sems.at[b % _NU]
        )

    for b in range(_LOOK):
        load(b).start()
    for b in range(_B):
        if b + _LOOK < _B:
            load(b + _LOOK).start()
        load(b).wait()
        u = ubuf[b % _NU]
        w2, b2 = _routing(u, wp_ref, bp_ref, wr_ref, br_ref)
        s = b % _NO
        if b >= _NO:
            pltpu.make_async_copy(
                obuf.at[s], o_pair.at[b - _NO], out_sems.at[s]
            ).wait()
        obuf[s] = jnp.dot(u, w2, preferred_element_type=jnp.float32) + b2
        pltpu.make_async_copy(obuf.at[s], o_pair.at[b], out_sems.at[s]).start()
    for b in range(_B - _NO, _B):
        pltpu.make_async_copy(
            obuf.at[b % _NO], o_pair.at[b], out_sems.at[b % _NO]
        ).wait()


def kernel(u_t, W_prim, b_prim, W_router, b_router):
    B, N, C = u_t.shape
    br = b_router.reshape(1, _NUM_P)
    u2 = u_t.reshape(B, N // 2, 2 * C)
    out2 = pl.pallas_call(
        _body,
        grid=(1,),
        in_specs=[
            pl.BlockSpec(memory_space=pltpu.MemorySpace.HBM),
            pl.BlockSpec((_NUM_P, C, _OUT_C), lambda g: (0, 0, 0)),
            pl.BlockSpec((_NUM_P, _OUT_C), lambda g: (0, 0)),
            pl.BlockSpec((C, _NUM_P), lambda g: (0, 0)),
            pl.BlockSpec((1, _NUM_P), lambda g: (0, 0)),
        ],
        out_specs=pl.BlockSpec(memory_space=pltpu.MemorySpace.HBM),
        out_shape=jax.ShapeDtypeStruct((B, N // 2, 2 * _OUT_C), jnp.float32),
        scratch_shapes=[
            pltpu.VMEM((_NU, N // 2, 2 * C), jnp.float32),
            pltpu.VMEM((_NO, N // 2, 2 * _OUT_C), jnp.float32),
            pltpu.SemaphoreType.DMA((_NU,)),
            pltpu.SemaphoreType.DMA((_NO,)),
        ],
    )(u2, W_prim, b_prim, W_router, br)
    return out2.reshape(B, N, _OUT_C)


# speedup vs baseline: 4.8835x; 4.8835x over previous
import jax
import jax.numpy as jnp
from jax.experimental import pallas as pl

_NUM_P = 8
_OUT_C = 64


def _step(u_ref, wpt_ref, bpt_ref, wrt_ref, brt_ref, out_ref):
    u = u_ref[0]
    pooled = jnp.sum(u, axis=1, keepdims=True) * (1.0 / u.shape[1])
    logits = (
        jnp.dot(wrt_ref[...], pooled, preferred_element_type=jnp.float32)
        + brt_ref[...]
    )
    iota = jax.lax.broadcasted_iota(jnp.int32, (_NUM_P, 1), 0)
    m1 = jnp.max(logits, axis=0, keepdims=True)
    i1 = jnp.min(jnp.where(logits == m1, iota, _NUM_P), axis=0, keepdims=True)
    masked = jnp.where(iota == i1, -jnp.inf, logits)
    m2 = jnp.max(masked, axis=0, keepdims=True)
    i2 = jnp.min(jnp.where(masked == m2, iota, _NUM_P), axis=0, keepdims=True)
    e = jnp.exp(m2 - m1)
    p1 = 1.0 / (1.0 + e)
    p2 = e / (1.0 + e)
    acc = jnp.eye(_OUT_C, dtype=jnp.float32)
    bacc = jnp.zeros((_OUT_C, 1), jnp.float32)
    for p in range(_NUM_P):
        w_p = jnp.where(i1 == p, p1, 0.0) + jnp.where(i2 == p, p2, 0.0)
        acc = acc + w_p * wpt_ref[p]
        bacc = bacc + w_p * bpt_ref[:, p : p + 1]
    out_ref[0] = (
        jnp.dot(acc, u, preferred_element_type=jnp.float32) + bacc
    )


def kernel(u_t, W_prim, b_prim, W_router, b_router):
    B, N, C = u_t.shape
    u_T = jnp.transpose(u_t, (0, 2, 1))
    wpt = jnp.transpose(W_prim, (0, 2, 1))
    bpt = jnp.transpose(b_prim)
    wrt = jnp.transpose(W_router)
    brt = b_router.reshape(_NUM_P, 1)
    out_T = pl.pallas_call(
        _step,
        grid=(B,),
        in_specs=[
            pl.BlockSpec((1, C, N), lambda b: (b, 0, 0)),
            pl.BlockSpec((_NUM_P, _OUT_C, C), lambda b: (0, 0, 0)),
            pl.BlockSpec((_OUT_C, _NUM_P), lambda b: (0, 0)),
            pl.BlockSpec((_NUM_P, C), lambda b: (0, 0)),
            pl.BlockSpec((_NUM_P, 1), lambda b: (0, 0)),
        ],
        out_specs=pl.BlockSpec((1, _OUT_C, N), lambda b: (b, 0, 0)),
        out_shape=jax.ShapeDtypeStruct((B, _OUT_C, N), jnp.float32),
    )(u_T, wpt, bpt, wrt, brt)
    return jnp.transpose(out_T, (0, 2, 1))


# speedup vs baseline: 6.3377x; 1.2978x over previous
import jax
import jax.numpy as jnp
from jax.experimental import pallas as pl
from jax.experimental.pallas import tpu as pltpu

_NUM_P = 8
_OUT_C = 64
_B = 16
_LOOK = 4
_NU = 6
_NO = 3

_DOT_T = (((0,), (0,)), ((), ()))


def _routing(u, wp_ref, bpt_ref, wr_ref, br_ref):
    pooled = jnp.sum(u, axis=1, keepdims=True) * (1.0 / u.shape[1])
    logits = (
        jax.lax.dot_general(
            wr_ref[...], pooled, _DOT_T, preferred_element_type=jnp.float32
        )
        + br_ref[...]
    )
    iota = jax.lax.broadcasted_iota(jnp.int32, (_NUM_P, 1), 0)
    m1 = jnp.max(logits, axis=0, keepdims=True)
    i1 = jnp.min(jnp.where(logits == m1, iota, _NUM_P), axis=0, keepdims=True)
    masked = jnp.where(iota == i1, -jnp.inf, logits)
    m2 = jnp.max(masked, axis=0, keepdims=True)
    i2 = jnp.min(jnp.where(masked == m2, iota, _NUM_P), axis=0, keepdims=True)
    e = jnp.exp(m2 - m1)
    p1 = 1.0 / (1.0 + e)
    p2 = e / (1.0 + e)
    acc = jnp.eye(_OUT_C, dtype=jnp.float32)
    bacc = jnp.zeros((_OUT_C, 1), jnp.float32)
    for p in range(_NUM_P):
        w_p = jnp.where(i1 == p, p1, 0.0) + jnp.where(i2 == p, p2, 0.0)
        acc = acc + w_p * wp_ref[p]
        bacc = bacc + w_p * bpt_ref[:, p : p + 1]
    return acc, bacc


def _body(u_hbm, wp_ref, bpt_ref, wr_ref, br_ref, out_hbm,
          ubuf, obuf, in_sems, out_sems):
    def load(b):
        return pltpu.make_async_copy(
            u_hbm.at[b], ubuf.at[b % _NU], in_sems.at[b % _NU]
        )

    for b in range(_LOOK):
        load(b).start()
    for b in range(_B):
        if b + _LOOK < _B:
            load(b + _LOOK).start()
        load(b).wait()
        u = ubuf[b % _NU]
        acc, bacc = _routing(u, wp_ref, bpt_ref, wr_ref, br_ref)
        s = b % _NO
        if b >= _NO:
            pltpu.make_async_copy(
                obuf.at[s], out_hbm.at[b - _NO], out_sems.at[s]
            ).wait()
        obuf[s] = (
            jax.lax.dot_general(
                acc, u, _DOT_T, preferred_element_type=jnp.float32
            )
            + bacc
        )
        pltpu.make_async_copy(obuf.at[s], out_hbm.at[b], out_sems.at[s]).start()
    for b in range(_B - _NO, _B):
        pltpu.make_async_copy(
            obuf.at[b % _NO], out_hbm.at[b], out_sems.at[b % _NO]
        ).wait()


def kernel(u_t, W_prim, b_prim, W_router, b_router):
    B, N, C = u_t.shape
    u_T = jnp.transpose(u_t, (0, 2, 1))
    bpt = jnp.transpose(b_prim)
    br = b_router.reshape(_NUM_P, 1)
    out_T = pl.pallas_call(
        _body,
        grid=(1,),
        in_specs=[
            pl.BlockSpec(memory_space=pltpu.MemorySpace.HBM),
            pl.BlockSpec((_NUM_P, C, _OUT_C), lambda g: (0, 0, 0)),
            pl.BlockSpec((_OUT_C, _NUM_P), lambda g: (0, 0)),
            pl.BlockSpec((C, _NUM_P), lambda g: (0, 0)),
            pl.BlockSpec((_NUM_P, 1), lambda g: (0, 0)),
        ],
        out_specs=pl.BlockSpec(memory_space=pltpu.MemorySpace.HBM),
        out_shape=jax.ShapeDtypeStruct((B, _OUT_C, N), jnp.float32),
        scratch_shapes=[
            pltpu.VMEM((_NU, C, N), jnp.float32),
            pltpu.VMEM((_NO, _OUT_C, N), jnp.float32),
            pltpu.SemaphoreType.DMA((_NU,)),
            pltpu.SemaphoreType.DMA((_NO,)),
        ],
    )(u_T, W_prim, bpt, W_router, br)
    return jnp.transpose(out_T, (0, 2, 1))
